# baseline (speedup 1.0000x reference)
import jax
import jax.numpy as jnp
from jax import lax
from jax.experimental import pallas as pl
from jax.experimental.pallas import tpu as pltpu
from jax.experimental.pallas import tpu_sc as plsc

N_NODES_C = 10000
N_EDGES_C = 320000
D_FEAT_C = 128
N_HOPS_C = 3

HALF = D_FEAT_C // 2
N_SUBCORES = 16
EDGES_PER_TILE = N_EDGES_C // N_SUBCORES
CHUNK = 80
N_CHUNKS = EDGES_PER_TILE // CHUNK
NBUF = 5
W_SMALL = 624
W_LAST = N_NODES_C - 15 * W_SMALL


def _sc_body(embed, packed_hbm, trend_hbm, zeros_hbm,
             out2d, cur,
             acc, gbufs, pbufs, tbufs,
             sem_idx, sem_g, sem_sc):
    c = lax.axis_index("c")
    s = lax.axis_index("s")
    row_off = c * N_NODES_C
    fcol = c * HALF

    nr0 = s * W_SMALL
    nrows_small = W_SMALL

    def idx_start(q, b):
        chunk_id = s * N_CHUNKS + q
        pltpu.make_async_copy(packed_hbm.at[chunk_id],
                              pbufs.at[b], sem_idx.at[b]).start()
        base = pl.multiple_of(s * EDGES_PER_TILE + q * CHUNK, 8)
        pltpu.make_async_copy(trend_hbm.at[pl.ds(base, CHUNK)],
                              tbufs.at[b], sem_idx.at[b]).start()

    def idx_wait(b):
        pltpu.make_async_copy(packed_hbm.at[0],
                              pbufs.at[b], sem_idx.at[b]).wait()
        pltpu.make_async_copy(trend_hbm.at[pl.ds(0, CHUNK)],
                              tbufs.at[b], sem_idx.at[b]).wait()

    def rowfix(b):
        for v in range(CHUNK // 16):
            pbufs[b, pl.ds(v * 16, 16)] = (
                pbufs[b, pl.ds(v * 16, 16)] + row_off)

    def gather_start(b):
        pltpu.make_async_copy(cur.at[pbufs.at[b, pl.ds(0, CHUNK)]],
                              gbufs.at[b], sem_g.at[b]).start()

    def gather_wait(b):
        pltpu.make_async_copy(cur.at[pbufs.at[b, pl.ds(0, CHUNK)]],
                              gbufs.at[b], sem_g.at[b]).wait()

    def scat_start(b):
        pltpu.make_async_copy(gbufs.at[b],
                              acc.at[pbufs.at[b, pl.ds(CHUNK, CHUNK)]],
                              sem_sc.at[b]).start(add=True)

    def scat_wait(b):
        pltpu.make_async_copy(gbufs.at[b],
                              acc.at[pbufs.at[b, pl.ds(CHUNK, CHUNK)]],
                              sem_sc.at[b]).wait()

    def make_scale(b):
        def scale_group(g, _):
            tv16 = tbufs[b, pl.ds(g * 16, 16)]
            for sub in range(2):
                e0 = g * 16 + sub * 8
                tvs = [jnp.full((16,), tv16[sub * 8 + l], jnp.float32)
                       for l in range(8)]
                vs = [[gbufs[b, e0 + l, pl.ds(jj * 16, 16)]
                       for jj in range(HALF // 16)] for l in range(8)]
                for l in range(8):
                    for jj in range(HALF // 16):
                        gbufs[b, e0 + l, pl.ds(jj * 16, 16)] = (
                            vs[l][jj] * tvs[l])
            return 0
        return scale_group

    scales = [make_scale(b) for b in range(NBUF)]

    @pl.when(s < 15)
    def _():
        pltpu.sync_copy(embed.at[pl.ds(nr0, W_SMALL), pl.ds(fcol, HALF)],
                        cur.at[pl.ds(row_off + nr0, W_SMALL)])
        pltpu.sync_copy(embed.at[pl.ds(nr0, W_SMALL), pl.ds(fcol, HALF)],
                        out2d.at[pl.ds(nr0, W_SMALL), pl.ds(fcol, HALF)])

    @pl.when(s == 15)
    def _():
        pltpu.sync_copy(
            embed.at[pl.ds(15 * W_SMALL, W_LAST), pl.ds(fcol, HALF)],
            cur.at[pl.ds(row_off + 15 * W_SMALL, W_LAST)])
        pltpu.sync_copy(
            embed.at[pl.ds(15 * W_SMALL, W_LAST), pl.ds(fcol, HALF)],
            out2d.at[pl.ds(15 * W_SMALL, W_LAST), pl.ds(fcol, HALF)])

    def hop_body(h, _):
        @pl.when(s < 15)
        def _():
            pltpu.sync_copy(zeros_hbm.at[pl.ds(0, W_SMALL)],
                            acc.at[pl.ds(s * W_SMALL, W_SMALL)])

        @pl.when(s == 15)
        def _():
            pltpu.sync_copy(zeros_hbm, acc.at[pl.ds(15 * W_SMALL, W_LAST)])

        plsc.subcore_barrier()

        idx_start(0, 0)
        idx_start(1, 1)
        idx_wait(0)
        rowfix(0)
        gather_start(0)

        def outer(jo, _):
            for b in range(NBUF):
                q = jo * NBUF + b
                bp = (b + 2) % NBUF
                bn = (b + 1) % NBUF

                @pl.when(q + 2 < N_CHUNKS)
                def _():
                    @pl.when(q + 2 >= NBUF)
                    def _():
                        scat_wait(bp)
                    idx_start(q + 2, bp)

                @pl.when(q + 1 < N_CHUNKS)
                def _():
                    idx_wait(bn)
                    rowfix(bn)
                    gather_start(bn)

                gather_wait(b)
                lax.fori_loop(0, CHUNK // 16, scales[b], 0)
                scat_start(b)
            return 0

        lax.fori_loop(0, N_CHUNKS // NBUF, outer, 0)
        for b in range(NBUF):
            scat_wait(b)
        plsc.subcore_barrier()

        ocol = (h + 1) * D_FEAT_C + fcol

        @pl.when(s < 15)
        def _():
            pltpu.sync_copy(acc.at[pl.ds(s * W_SMALL, W_SMALL)],
                            cur.at[pl.ds(row_off + s * W_SMALL, W_SMALL)])
            pltpu.sync_copy(acc.at[pl.ds(s * W_SMALL, W_SMALL)],
                            out2d.at[pl.ds(s * W_SMALL, W_SMALL),
                                     pl.ds(ocol, HALF)])

        @pl.when(s == 15)
        def _():
            pltpu.sync_copy(acc.at[pl.ds(15 * W_SMALL, W_LAST)],
                            cur.at[pl.ds(row_off + 15 * W_SMALL, W_LAST)])
            pltpu.sync_copy(acc.at[pl.ds(15 * W_SMALL, W_LAST)],
                            out2d.at[pl.ds(15 * W_SMALL, W_LAST),
                                     pl.ds(ocol, HALF)])

        plsc.subcore_barrier()
        return 0

    lax.fori_loop(0, N_HOPS_C, hop_body, 0)


@jax.jit
def _sc_call(embed, packed, trend, zeros):
    out_t = [
        jax.ShapeDtypeStruct((N_NODES_C, (N_HOPS_C + 1) * D_FEAT_C),
                             jnp.float32),
        jax.ShapeDtypeStruct((2 * N_NODES_C, HALF), jnp.float32),
    ]
    mesh = plsc.VectorSubcoreMesh(core_axis_name="c", subcore_axis_name="s")
    f = pl.kernel(
        _sc_body,
        out_type=out_t,
        mesh=mesh,
        compiler_params=pltpu.CompilerParams(use_tc_tiling_on_sc=False),
        scratch_types=[
            pltpu.VMEM_SHARED((N_NODES_C, HALF), jnp.float32),
            pltpu.VMEM((NBUF, CHUNK, HALF), jnp.float32),
            pltpu.VMEM((NBUF, 2 * CHUNK), jnp.int32),
            pltpu.VMEM((NBUF, CHUNK), jnp.float32),
            pltpu.SemaphoreType.DMA((NBUF,)),
            pltpu.SemaphoreType.DMA((NBUF,)),
            pltpu.SemaphoreType.DMA((NBUF,)),
        ],
    )
    return f(embed, packed, trend, zeros)


def kernel(embed, adj_sp_norm, edge_index, edge_weight, trend):
    del adj_sp_norm, edge_weight
    row = edge_index[0].astype(jnp.int32)
    col = edge_index[1].astype(jnp.int32)
    packed = jnp.concatenate(
        [row.reshape(-1, CHUNK), col.reshape(-1, CHUNK)], axis=1)
    zeros = jnp.zeros((W_LAST, HALF), jnp.float32)
    out2d, _ = _sc_call(embed, packed, trend, zeros)
    return out2d.reshape(N_NODES_C, N_HOPS_C + 1, D_FEAT_C)
